# baseline (speedup 1.0000x reference)
import functools

import jax
import jax.numpy as jnp
from jax import lax
from jax.experimental import pallas as pl
from jax.experimental.pallas import tpu as pltpu
from jax.experimental.pallas import tpu_sc as plsc

T = 8
COMPANIES = 100000
POSITIONS = 100000
D = 64
B = 16384

NC = 2
NS = 16
NW = NC * NS
NQ = 196
QPW = 7
CAP = 2048
QW = 512
RING = 32
INFLIGHT = 24


def _splat(s):
    return lax.broadcast_in_dim(jnp.int32(s) if isinstance(s, int) else s,
                                (16,), ())


def _iota():
    return lax.iota(jnp.int32, 16)


def _sc_body(c_hbm, p_hbm, te_hbm, com_hbm, pos_hbm,
             out_com_hbm, out_pos_hbm,
             c_v, p_v, te_v, tqc_v, blc_v, tqp_v, blp_v,
             slab0_v, slab1_v, stage_c_v, stage_p_v,
             sem_in, sem_s0, sem_s1, sem_oc, sem_op):
    wid = lax.axis_index("s") * NC + lax.axis_index("c")
    wid_s = _splat(wid)

    PADW = 100096

    def fetch(tbl_hbm, t, off, slot_ref, sem):
        pltpu.make_async_copy(
            tbl_hbm.at[pl.ds(t * D, D), pl.ds(off, QW)],
            slot_ref, sem).start()

    fetch(com_hbm, 0, wid * QW, slab0_v, sem_s0)
    fetch(com_hbm, 1, wid * QW, slab1_v, sem_s1)

    cp_c = pltpu.make_async_copy(c_hbm, c_v, sem_in)
    cp_p = pltpu.make_async_copy(p_hbm, p_v, sem_in)
    cp_t = pltpu.make_async_copy(te_hbm, te_v, sem_in)
    cp_c.start(); cp_p.start(); cp_t.start()
    cp_c.wait(); cp_p.wait(); cp_t.wait()

    def pbody(i, carry):
        cc_v, cpn_v = carry
        sl = pl.ds(i * 16, 16)
        cv = c_v[sl]
        pv = p_v[sl]
        tev = te_v[sl]
        bv = _splat(i * 16) + _iota()

        qc = lax.shift_right_logical(cv, 9)
        mc = (qc & 31) == wid_s
        rank = plsc.cumsum(mc.astype(jnp.int32)) - 1
        plsc.store_scatter(tqc_v, [cc_v + rank], qc * 8 + tev, mask=mc)
        plsc.store_scatter(blc_v, [cc_v + rank], bv * QW + (cv & (QW - 1)),
                           mask=mc)
        cc_v = cc_v + plsc.all_reduce_population_count(mc)

        qp = lax.shift_right_logical(pv, 9)
        mp = (qp & 31) == wid_s
        rankp = plsc.cumsum(mp.astype(jnp.int32)) - 1
        plsc.store_scatter(tqp_v, [cpn_v + rankp], qp * 8 + tev, mask=mp)
        plsc.store_scatter(blp_v, [cpn_v + rankp], bv * QW + (pv & (QW - 1)),
                           mask=mp)
        cpn_v = cpn_v + plsc.all_reduce_population_count(mp)
        return (cc_v, cpn_v)

    zeros = _splat(0)
    cc_v, cpn_v = lax.fori_loop(0, B // 16, pbody, (zeros, zeros))
    cnt_c = jnp.max(cc_v)
    cnt_p = jnp.max(cpn_v)

    def slab_wait(sem, slot_ref):
        pltpu.make_async_copy(
            com_hbm.at[pl.ds(0, D), pl.ds(0, QW)], slot_ref, sem).wait()

    def scan(slot_ref, tq_list, bl_list, cnt, tq_tgt, delta, out_ref, sem_out,
             stage_v):
        cnt_s = _splat(cnt)
        tgt_s = _splat(tq_tgt)
        nk = lax.shift_right_logical(cnt + 15, 4)

        def kbody(k, o_in):
            sl = pl.ds(k * 16, 16)
            tqv = tq_list[sl]
            lane_ok = (_splat(k * 16) + _iota()) < cnt_s
            m = (tqv == tgt_s) & lane_ok

            def wcond(carry):
                m_, _o = carry
                return jnp.max(m_.astype(jnp.int32)) > 0

            def wbody(carry):
                m_, o_ = carry
                blv = bl_list[sl]
                ffs = plsc.all_reduce_ffs(m_)
                sel = _iota() == ffs
                bl_s = jnp.max(jnp.where(sel, blv, 0))
                b = lax.shift_right_logical(bl_s, 9)
                l = (bl_s & (QW - 1)) + delta
                slot = o_ & (RING - 1)
                for kk in range(4):
                    rvec = _splat(kk * 16) + _iota()
                    vals = plsc.load_gather(slot_ref, [rvec, _splat(l)])
                    stage_v[pl.ds(slot * 64 + kk * 16, 16)] = vals

                @pl.when(o_ >= INFLIGHT)
                def _():
                    pltpu.make_async_copy(
                        stage_v.at[pl.ds(0, 64)],
                        out_ref.at[pl.ds(0, 64)], sem_out).wait()

                pltpu.make_async_copy(
                    stage_v.at[pl.ds(slot * 64, 64)],
                    out_ref.at[pl.ds(b * 64, 64)], sem_out).start()
                return (m_ & (~sel), o_ + 1)

            _m, o_out = lax.while_loop(wcond, wbody, (m, o_in))
            return o_out

        return lax.fori_loop(0, nk, kbody, jnp.int32(0))

    def qbody(jq, carry):
        oc_c, oc_p = carry
        q = wid + jq * 32
        qn = q + 32
        valid = q < NQ
        validn = qn < NQ
        off = jnp.minimum(q * QW, PADW - QW)
        offn = jnp.minimum(qn * QW, PADW - QW)
        delta = q * QW - off

        for st in range(16):
            t = st & 7
            slot_ref = slab0_v if (st & 1) == 0 else slab1_v
            sem = sem_s0 if (st & 1) == 0 else sem_s1

            @pl.when(valid)
            def _(sem=sem, slot_ref=slot_ref):
                slab_wait(sem, slot_ref)

            if st < 8:
                n = scan(slot_ref, tqc_v, blc_v, cnt_c, q * 8 + t, delta,
                         out_com_hbm, sem_oc, stage_c_v)
                oc_c = oc_c + n
            else:
                n = scan(slot_ref, tqp_v, blp_v, cnt_p, q * 8 + t, delta,
                         out_pos_hbm, sem_op, stage_p_v)
                oc_p = oc_p + n

            stn = st + 2
            if stn < 16:
                tbl_hbm = com_hbm if stn < 8 else pos_hbm
                tn = stn & 7

                @pl.when(valid)
                def _(tbl_hbm=tbl_hbm, tn=tn, slot_ref=slot_ref, sem=sem):
                    fetch(tbl_hbm, tn, off, slot_ref, sem)
            else:
                tn = stn & 7

                @pl.when(validn)
                def _(tn=tn, slot_ref=slot_ref, sem=sem):
                    fetch(com_hbm, tn, offn, slot_ref, sem)

        return (oc_c, oc_p)

    oc_c, oc_p = lax.fori_loop(0, QPW, qbody, (jnp.int32(0), jnp.int32(0)))

    def drain(n, out_ref, sem, stage_v):
        def db(i, _):
            pltpu.make_async_copy(
                stage_v.at[pl.ds(0, 64)],
                out_ref.at[pl.ds(0, 64)], sem).wait()
            return 0
        lax.fori_loop(0, n, db, 0)

    drain(jnp.minimum(oc_c, INFLIGHT), out_com_hbm, sem_oc, stage_c_v)
    drain(jnp.minimum(oc_p, INFLIGHT), out_pos_hbm, sem_op, stage_p_v)


@jax.jit
def _sc_gather(c, p, t_e, com2d, pos2d):
    mesh = plsc.VectorSubcoreMesh(core_axis_name="c", subcore_axis_name="s",
                                  num_cores=NC, num_subcores=NS)
    return pl.kernel(
        _sc_body,
        out_type=(jax.ShapeDtypeStruct((B * D,), jnp.float32),
                  jax.ShapeDtypeStruct((B * D,), jnp.float32)),
        mesh=mesh,
        compiler_params=pltpu.CompilerParams(use_tc_tiling_on_sc=True,
                                             disable_bounds_checks=True,
                                             needs_layout_passes=False),
        scratch_types=[
            pltpu.VMEM((B,), jnp.int32),
            pltpu.VMEM((B,), jnp.int32),
            pltpu.VMEM((B,), jnp.int32),
            pltpu.VMEM((CAP,), jnp.int32),
            pltpu.VMEM((CAP,), jnp.int32),
            pltpu.VMEM((CAP,), jnp.int32),
            pltpu.VMEM((CAP,), jnp.int32),
            pltpu.VMEM((D, QW), jnp.float32),
            pltpu.VMEM((D, QW), jnp.float32),
            pltpu.VMEM((RING * D,), jnp.float32),
            pltpu.VMEM((RING * D,), jnp.float32),
            pltpu.SemaphoreType.DMA,
            pltpu.SemaphoreType.DMA,
            pltpu.SemaphoreType.DMA,
            pltpu.SemaphoreType.DMA,
            pltpu.SemaphoreType.DMA,
        ],
    )(c, p, t_e, com2d, pos2d)


def kernel(c, p, t_s, t_e, com_embs, pos_embs):
    del t_s
    com2d = com_embs.transpose(0, 2, 1).reshape(T * D, COMPANIES)
    pos2d = pos_embs.transpose(0, 2, 1).reshape(T * D, POSITIONS)
    out_com, out_pos = _sc_gather(c, p, t_e, com2d, pos2d)
    return (out_com.reshape(B, D), out_pos.reshape(B, D))
